# Initial kernel scaffold; baseline (speedup 1.0000x reference)
import jax
import jax.numpy as jnp
from jax.experimental import pallas as pl


def kernel(seq, W, U):
    raise NotImplementedError("write your pallas kernel here")


# speedup vs baseline: 2.5280x; 2.5280x over previous
import functools

import jax
import jax.numpy as jnp
from jax import lax
from jax.experimental import pallas as pl
from jax.experimental.pallas import tpu as pltpu
from jax.experimental.pallas import tpu_sc as plsc

VOCAB = 100000
EMB = 64
REGION = 3
B, L = 1024, 200

NC, NS, LANES = 2, 16, 16
NW = NC * NS
ROWS_PER_W = B // NW
LP = 208
SPLIT = 112
NCHUNK = LP // LANES


def _region_kernel(seq_hbm, w_hbm, u3_hbm, out_hbm,
                   seq_v, idx_a, idx_b, m_v, w_rows, u_rows, h_rows, sem):
    wid = lax.axis_index("s") * NC + lax.axis_index("c")

    seq_v[pl.ds(192, 16)] = jnp.zeros((16,), jnp.int32)
    m_v[pl.ds(LP, 16)] = jnp.zeros((16,), jnp.float32)

    @pl.loop(0, ROWS_PER_W)
    def _row(j):
        row = wid * ROWS_PER_W + j
        pltpu.sync_copy(seq_hbm.at[pl.ds(row * L, L)], seq_v.at[pl.ds(0, L)])

        for k in range(NCHUNK):
            s = seq_v[pl.ds(k * 16, 16)]
            idx = jnp.maximum(s, 1) - 1
            if k * 16 < SPLIT:
                idx_a[pl.ds(k * 16, 16)] = idx
            else:
                idx_b[pl.ds(k * 16 - SPLIT, 16)] = idx
            m_v[pl.ds(k * 16, 16)] = jnp.where(
                s != 0, jnp.float32(1.0), jnp.float32(0.0))

        c1 = pltpu.async_copy(w_hbm.at[idx_a], w_rows.at[pl.ds(0, SPLIT)], sem)
        c2 = pltpu.async_copy(w_hbm.at[idx_b],
                              w_rows.at[pl.ds(SPLIT, LP - SPLIT)], sem)
        c3 = pltpu.async_copy(u3_hbm.at[idx_a], u_rows.at[pl.ds(0, SPLIT)], sem)
        c4 = pltpu.async_copy(u3_hbm.at[idx_b],
                              u_rows.at[pl.ds(SPLIT, LP - SPLIT)], sem)
        c1.wait()
        c2.wait()
        c3.wait()
        c4.wait()

        mm0 = m_v[pl.ds(0, 16)]
        m0 = mm0[0]
        mr0 = mm0[1]
        zero = jnp.zeros((16,), jnp.float32)
        for c in range(EMB // 16):
            w = w_rows[0, pl.ds(c * 16, 16)] * m0
            p1 = u_rows[0, pl.ds(EMB + c * 16, 16)] * w
            p2 = u_rows[1, pl.ds(2 * EMB + c * 16, 16)] * w * mr0
            h_rows[pl.ds(c * 16, 16)] = jnp.maximum(
                jnp.maximum(p1, p2), zero)

        @pl.loop(1, L)
        def _token(l):
            mm = m_v[pl.ds(l - 1, 16)]
            ml = mm[0]
            mc = mm[1]
            mr = mm[2]
            for c in range(EMB // 16):
                w = w_rows[l, pl.ds(c * 16, 16)] * mc
                p0 = u_rows[l - 1, pl.ds(c * 16, 16)] * w * ml
                p1 = u_rows[l, pl.ds(EMB + c * 16, 16)] * w
                p2 = u_rows[l + 1, pl.ds(2 * EMB + c * 16, 16)] * w * mr
                h_rows[pl.ds(l * EMB + c * 16, 16)] = jnp.maximum(
                    jnp.maximum(p0, p1), p2)

        pltpu.sync_copy(h_rows, out_hbm.at[pl.ds(row * L * EMB, L * EMB)])


@jax.jit
def _run(seq, W, U3):
    mesh = plsc.VectorSubcoreMesh(
        core_axis_name="c", subcore_axis_name="s",
        num_cores=NC, num_subcores=NS)
    kfn = pl.kernel(
        _region_kernel,
        out_type=jax.ShapeDtypeStruct((B * L * EMB,), jnp.float32),
        mesh=mesh,
        compiler_params=pltpu.CompilerParams(use_tc_tiling_on_sc=False),
        scratch_types=[
            pltpu.VMEM((LP,), jnp.int32),
            pltpu.VMEM((SPLIT,), jnp.int32),
            pltpu.VMEM((LP - SPLIT,), jnp.int32),
            pltpu.VMEM((LP + 16,), jnp.float32),
            pltpu.VMEM((LP, EMB), jnp.float32),
            pltpu.VMEM((LP, REGION * EMB), jnp.float32),
            pltpu.VMEM((L * EMB,), jnp.float32),
            pltpu.SemaphoreType.DMA,
        ],
    )
    return kfn(seq, W, U3)


def kernel(seq, W, U):
    seq = seq.astype(jnp.int32).reshape(B * L)
    U3 = U.reshape(VOCAB - 1, REGION * EMB)
    return _run(seq, W, U3).reshape(B, L, EMB)
